# Initial kernel scaffold; baseline (speedup 1.0000x reference)
import functools

import jax
import jax.numpy as jnp
from jax import lax
from jax.experimental import pallas as pl
from jax.experimental.pallas import tpu as pltpu
from jax.experimental.pallas import tpu_sc as plsc

_NC = 2
_NS = 16
_NW = _NC * _NS

_CHUNK = 128
_NBUF = 2


def _emb_call(n_idx, d, n_ch):
    mesh = plsc.VectorSubcoreMesh(core_axis_name="c", subcore_axis_name="s")
    b_per_w = n_ch * _CHUNK

    @functools.partial(
        pl.kernel,
        mesh=mesh,
        out_type=jax.ShapeDtypeStruct((n_idx, d), jnp.float32),
        scratch_types=[
            pltpu.VMEM((n_ch, _CHUNK), jnp.int32),
            pltpu.VMEM((_NBUF, _CHUNK, d), jnp.float32),
            pltpu.SemaphoreType.DMA,
            pltpu.SemaphoreType.DMA,
        ],
    )
    def emb(idx_hbm, table_hbm, out_hbm, idx_v, rows_v, gsem0, gsem1):
        gsems = (gsem0, gsem1)
        wid = lax.axis_index("s") * _NC + lax.axis_index("c")
        base = wid * b_per_w
        pltpu.sync_copy(idx_hbm.at[pl.ds(wid * n_ch, n_ch)], idx_v)

        for b in range(_NBUF):
            pltpu.async_copy(table_hbm.at[idx_v.at[b]], rows_v.at[b], gsems[b])

        def body(step, carry):
            i = step * _NBUF
            for b in range(_NBUF):
                g = i + b
                pltpu.make_async_copy(
                    table_hbm.at[idx_v.at[b]], rows_v.at[b], gsems[b]
                ).wait()
                pltpu.sync_copy(
                    rows_v.at[b], out_hbm.at[pl.ds(base + g * _CHUNK, _CHUNK)]
                )
                pltpu.async_copy(
                    table_hbm.at[idx_v.at[g + _NBUF]], rows_v.at[b], gsems[b]
                )
            return carry

        lax.fori_loop(0, (n_ch - _NBUF) // _NBUF, body, 0)

        for b in range(_NBUF):
            g = n_ch - _NBUF + b
            pltpu.make_async_copy(
                table_hbm.at[idx_v.at[b]], rows_v.at[b], gsems[b]
            ).wait()
            pltpu.sync_copy(
                rows_v.at[b], out_hbm.at[pl.ds(base + g * _CHUNK, _CHUNK)]
            )

    return emb


def kernel(city, table):
    b0, b1 = city.shape
    v, d = table.shape
    n_idx = b0 * b1
    n_ch = n_idx // (_NW * _CHUNK)
    idx = city.reshape(_NW * n_ch, _CHUNK).astype(jnp.int32)
    out = _emb_call(n_idx, d, n_ch)(idx, table)
    return out.reshape(b0, b1, d)


# speedup vs baseline: 4.5440x; 4.5440x over previous
import functools

import jax
import jax.numpy as jnp
from jax import lax
from jax.experimental import pallas as pl
from jax.experimental.pallas import tpu as pltpu
from jax.experimental.pallas import tpu_sc as plsc

_NC = 2
_NS = 16
_NW = _NC * _NS

_CHUNK = 128
_NBUF = 2


def _emb_call(n_idx, d, n_ch):
    mesh = plsc.VectorSubcoreMesh(core_axis_name="c", subcore_axis_name="s")
    b_per_w = n_ch * _CHUNK

    @functools.partial(
        pl.kernel,
        mesh=mesh,
        out_type=jax.ShapeDtypeStruct((n_idx, d), jnp.float32),
        compiler_params=pltpu.CompilerParams(use_tc_tiling_on_sc=False),
        scratch_types=[
            pltpu.VMEM((n_ch, _CHUNK), jnp.int32),
            pltpu.VMEM((_NBUF, _CHUNK, d), jnp.float32),
            pltpu.SemaphoreType.DMA,
            pltpu.SemaphoreType.DMA,
        ],
    )
    def emb(idx_hbm, table_hbm, out_hbm, idx_v, rows_v, gsem0, gsem1):
        gsems = (gsem0, gsem1)
        wid = lax.axis_index("s") * _NC + lax.axis_index("c")
        base = wid * b_per_w
        pltpu.sync_copy(idx_hbm.at[wid], idx_v)

        for b in range(_NBUF):
            pltpu.async_copy(table_hbm.at[idx_v.at[b]], rows_v.at[b], gsems[b])

        def body(step, carry):
            i = step * _NBUF
            for b in range(_NBUF):
                g = i + b
                pltpu.make_async_copy(
                    table_hbm.at[idx_v.at[b]], rows_v.at[b], gsems[b]
                ).wait()
                pltpu.sync_copy(
                    rows_v.at[b], out_hbm.at[pl.ds(base + g * _CHUNK, _CHUNK)]
                )
                pltpu.async_copy(
                    table_hbm.at[idx_v.at[g + _NBUF]], rows_v.at[b], gsems[b]
                )
            return carry

        lax.fori_loop(0, (n_ch - _NBUF) // _NBUF, body, 0)

        for b in range(_NBUF):
            g = n_ch - _NBUF + b
            pltpu.make_async_copy(
                table_hbm.at[idx_v.at[b]], rows_v.at[b], gsems[b]
            ).wait()
            pltpu.sync_copy(
                rows_v.at[b], out_hbm.at[pl.ds(base + g * _CHUNK, _CHUNK)]
            )

    return emb


def kernel(city, table):
    b0, b1 = city.shape
    v, d = table.shape
    n_idx = b0 * b1
    n_ch = n_idx // (_NW * _CHUNK)
    idx = city.reshape(_NW, n_ch, _CHUNK).astype(jnp.int32)
    out = _emb_call(n_idx, d, n_ch)(idx, table)
    return out.reshape(b0, b1, d)


# speedup vs baseline: 4.6911x; 1.0324x over previous
import functools

import jax
import jax.numpy as jnp
from jax import lax
from jax.experimental import pallas as pl
from jax.experimental.pallas import tpu as pltpu
from jax.experimental.pallas import tpu_sc as plsc

_NC = 2
_NS = 16
_NW = _NC * _NS

_CHUNK = 256
_NBUF = 5


def _emb_call(n_idx, d, n_ch):
    mesh = plsc.VectorSubcoreMesh(core_axis_name="c", subcore_axis_name="s")
    b_per_w = n_ch * _CHUNK

    @functools.partial(
        pl.kernel,
        mesh=mesh,
        out_type=jax.ShapeDtypeStruct((n_idx, d), jnp.float32),
        compiler_params=pltpu.CompilerParams(use_tc_tiling_on_sc=False),
        scratch_types=[
            pltpu.VMEM((n_ch, _CHUNK), jnp.int32),
            pltpu.VMEM((_NBUF, _CHUNK, d), jnp.float32),
        ]
        + [pltpu.SemaphoreType.DMA] * _NBUF,
    )
    def emb(idx_hbm, table_hbm, out_hbm, idx_v, rows_v, *gsems):
        wid = lax.axis_index("s") * _NC + lax.axis_index("c")
        base = wid * b_per_w
        pltpu.sync_copy(idx_hbm.at[wid], idx_v)

        for b in range(_NBUF):
            pltpu.async_copy(table_hbm.at[idx_v.at[b]], rows_v.at[b], gsems[b])

        def body(step, carry):
            i = step * _NBUF
            for b in range(_NBUF):
                g = i + b
                pltpu.make_async_copy(
                    table_hbm.at[idx_v.at[b]], rows_v.at[b], gsems[b]
                ).wait()
                pltpu.sync_copy(
                    rows_v.at[b], out_hbm.at[pl.ds(base + g * _CHUNK, _CHUNK)]
                )
                pltpu.async_copy(
                    table_hbm.at[idx_v.at[g + _NBUF]], rows_v.at[b], gsems[b]
                )
            return carry

        lax.fori_loop(0, (n_ch - _NBUF) // _NBUF, body, 0)

        for b in range(_NBUF):
            g = n_ch - _NBUF + b
            pltpu.make_async_copy(
                table_hbm.at[idx_v.at[b]], rows_v.at[b], gsems[b]
            ).wait()
            pltpu.sync_copy(
                rows_v.at[b], out_hbm.at[pl.ds(base + g * _CHUNK, _CHUNK)]
            )

    return emb


def kernel(city, table):
    b0, b1 = city.shape
    v, d = table.shape
    n_idx = b0 * b1
    n_ch = n_idx // (_NW * _CHUNK)
    idx = city.reshape(_NW, n_ch, _CHUNK).astype(jnp.int32)
    out = _emb_call(n_idx, d, n_ch)(idx, table)
    return out.reshape(b0, b1, d)
